# Initial kernel scaffold; baseline (speedup 1.0000x reference)
import jax
import jax.numpy as jnp
from jax.experimental import pallas as pl


def kernel(query_features, query_xyz, key_value_features, key_value_xyz, W1, gamma1, beta1, W2, gamma2, beta2):
    raise NotImplementedError("write your pallas kernel here")


# speedup vs baseline: 4.4003x; 4.4003x over previous
import jax
import jax.numpy as jnp
from jax import lax
from jax.experimental import pallas as pl
from jax.experimental.pallas import tpu as pltpu

K = 16
EPS = 1e-5
BLOCK_Q = 128
BIG = 3.0e38


def _knn_body(qx_ref, kvxT_ref, kvf_ref, qf_ref, w1_ref,
              s1_ref, s2_ref, m_ref, q1_ref, d_ref):
    qx = qx_ref[0]
    kvxT = kvxT_ref[0]
    kvf = kvf_ref[0]
    qf = qf_ref[0]
    w1 = w1_ref[...]
    C = kvf.shape[1]
    w1a = w1[:, :C]
    w1d = w1[:, C:] - w1a
    A = jnp.dot(kvf, w1a.T, preferred_element_type=jnp.float32,
                precision=lax.Precision.HIGHEST)
    Q1 = jnp.dot(qf, w1d.T, preferred_element_type=jnp.float32,
                 precision=lax.Precision.HIGHEST)

    def rq(x):
        return x.astype(jnp.bfloat16).astype(jnp.float32)

    qx0, qx1, qx2 = qx[:, 0:1], qx[:, 1:2], qx[:, 2:3]
    kx0, kx1, kx2 = kvxT[0:1, :], kvxT[1:2, :], kvxT[2:3, :]
    qk = rq(qx0) * rq(kx0) + rq(qx1) * rq(kx1) + rq(qx2) * rq(kx2)
    q2 = qx0 * qx0 + qx1 * qx1 + qx2 * qx2
    k2 = kx0 * kx0 + kx1 * kx1 + kx2 * kx2
    d_ref[...] = (-2.0 * qk + q2) + k2

    def round_fn(_, carry):
        s1, s2, m = carry
        D = d_ref[...]
        v = jnp.min(D, axis=1, keepdims=True)
        msk = D == v
        row = jnp.dot(msk.astype(jnp.float32), A,
                      preferred_element_type=jnp.float32,
                      precision=lax.Precision.HIGHEST)
        d_ref[...] = jnp.where(msk, BIG, D)
        return s1 + row, s2 + row * row, jnp.maximum(m, row)

    s1, s2, m = lax.fori_loop(
        0, K,
        round_fn,
        (jnp.zeros_like(Q1), jnp.zeros_like(Q1), jnp.full_like(Q1, -BIG)))

    s1_ref[0] = s1
    s2_ref[0] = s2
    m_ref[0] = m
    q1_ref[0] = Q1


def _head_body(s1_ref, s2_ref, m_ref, q1_ref, w2_ref,
               g1_ref, b1_ref, g2_ref, b2_ref, out_ref):
    s1 = s1_ref[...]
    s2 = s2_ref[...]
    m = m_ref[...]
    q1 = q1_ref[...]
    w2 = w2_ref[...]
    n = s1.shape[0]
    kf = jnp.float32(K)
    cnt1 = jnp.float32(n * K)

    sum1 = jnp.sum(s1 + kf * q1, axis=0, keepdims=True)
    sumsq = jnp.sum(s2 + 2.0 * q1 * s1 + kf * q1 * q1, axis=0, keepdims=True)
    mu1 = sum1 / cnt1
    var1 = sumsq / cnt1 - mu1 * mu1
    rs1 = lax.rsqrt(var1 + EPS)
    y = (m + q1 - mu1) * rs1 * g1_ref[...] + b1_ref[...]
    y = jnp.where(y >= 0.0, y, 0.2 * y)

    z = jnp.dot(y, w2.T, preferred_element_type=jnp.float32,
                precision=lax.Precision.HIGHEST)
    mu2 = jnp.sum(z, axis=0, keepdims=True) / n
    var2 = jnp.sum(z * z, axis=0, keepdims=True) / n - mu2 * mu2
    o = (z - mu2) * lax.rsqrt(var2 + EPS) * g2_ref[...] + b2_ref[...]
    out_ref[...] = jnp.where(o >= 0.0, o, 0.2 * o)


def kernel(query_features, query_xyz, key_value_features, key_value_xyz,
           W1, gamma1, beta1, W2, gamma2, beta2):
    B, Nq, C = query_features.shape
    Nkv = key_value_features.shape[1]
    Cd = W1.shape[0]
    nqb = Nq // BLOCK_Q
    kvxT = jnp.transpose(key_value_xyz, (0, 2, 1))

    s1, s2, m, q1 = pl.pallas_call(
        _knn_body,
        grid=(B, nqb),
        in_specs=[
            pl.BlockSpec((1, BLOCK_Q, 3), lambda b, q: (b, q, 0)),
            pl.BlockSpec((1, 3, Nkv), lambda b, q: (b, 0, 0)),
            pl.BlockSpec((1, Nkv, C), lambda b, q: (b, 0, 0)),
            pl.BlockSpec((1, BLOCK_Q, C), lambda b, q: (b, q, 0)),
            pl.BlockSpec((Cd, 2 * C), lambda b, q: (0, 0)),
        ],
        out_specs=[
            pl.BlockSpec((1, BLOCK_Q, Cd), lambda b, q: (b, q, 0)),
            pl.BlockSpec((1, BLOCK_Q, Cd), lambda b, q: (b, q, 0)),
            pl.BlockSpec((1, BLOCK_Q, Cd), lambda b, q: (b, q, 0)),
            pl.BlockSpec((1, BLOCK_Q, Cd), lambda b, q: (b, q, 0)),
        ],
        out_shape=[jax.ShapeDtypeStruct((B, Nq, Cd), jnp.float32)] * 4,
        scratch_shapes=[pltpu.VMEM((BLOCK_Q, Nkv), jnp.float32)],
    )(query_xyz, kvxT, key_value_features, query_features, W1)

    n = B * Nq
    out = pl.pallas_call(
        _head_body,
        out_shape=jax.ShapeDtypeStruct((n, C), jnp.float32),
    )(s1.reshape(n, Cd), s2.reshape(n, Cd), m.reshape(n, Cd),
      q1.reshape(n, Cd), W2,
      gamma1.reshape(1, Cd), beta1.reshape(1, Cd),
      gamma2.reshape(1, C), beta2.reshape(1, C))

    return out.reshape(B, Nq, C)


# speedup vs baseline: 9.3689x; 2.1292x over previous
import jax
import jax.numpy as jnp
from jax import lax
from jax.experimental import pallas as pl
from jax.experimental.pallas import tpu as pltpu

K = 16
EPS = 1e-5
BLOCK_Q = 128
BIG = 3.0e38


def _proj_body(kvf_ref, w1_ref, ab_ref):
    w1 = w1_ref[...]
    C = kvf_ref.shape[2]
    w1a = w1[:, :C]
    A = jnp.dot(kvf_ref[0], w1a.T, preferred_element_type=jnp.float32,
                precision=lax.Precision.HIGHEST)
    hi = A.astype(jnp.bfloat16)
    lo = (A - hi.astype(jnp.float32)).astype(jnp.bfloat16)
    ab_ref[0] = jnp.concatenate([hi, lo], axis=1)


def _knn_body(qx_ref, kvxT_ref, ab_ref, qf_ref, w1_ref,
              s1_ref, s2_ref, m_ref, q1_ref, d_ref):
    qx = qx_ref[0]
    kvxT = kvxT_ref[0]
    Ab = ab_ref[0]
    qf = qf_ref[0]
    w1 = w1_ref[...]
    C = qf.shape[1]
    w1d = w1[:, C:] - w1[:, :C]
    Q1 = jnp.dot(qf, w1d.T, preferred_element_type=jnp.float32,
                 precision=lax.Precision.HIGHEST)

    def rq(x):
        return x.astype(jnp.bfloat16).astype(jnp.float32)

    qx0, qx1, qx2 = qx[:, 0:1], qx[:, 1:2], qx[:, 2:3]
    kx0, kx1, kx2 = kvxT[0:1, :], kvxT[1:2, :], kvxT[2:3, :]
    qk = rq(qx0) * rq(kx0) + rq(qx1) * rq(kx1) + rq(qx2) * rq(kx2)
    q2 = qx0 * qx0 + qx1 * qx1 + qx2 * qx2
    k2 = kx0 * kx0 + kx1 * kx1 + kx2 * kx2
    d_ref[...] = (-2.0 * qk + q2) + k2

    Cd = w1.shape[0]
    BQ, Nkv = d_ref.shape
    iota = lax.broadcasted_iota(jnp.int32, (BQ, Nkv), 1).astype(jnp.float32)

    def round_fn(_, carry):
        s1, s2, m = carry
        D = d_ref[...]
        v = jnp.min(D, axis=1, keepdims=True)
        cand = jnp.where(D == v, iota, BIG)
        jmin = jnp.min(cand, axis=1, keepdims=True)
        msk = cand == jmin
        row2 = jnp.dot(msk.astype(jnp.bfloat16), Ab,
                       preferred_element_type=jnp.float32)
        row = row2[:, :Cd] + row2[:, Cd:]
        d_ref[...] = jnp.where(msk, BIG, D)
        return s1 + row, s2 + row * row, jnp.maximum(m, row)

    s1, s2, m = lax.fori_loop(
        0, K,
        round_fn,
        (jnp.zeros_like(Q1), jnp.zeros_like(Q1), jnp.full_like(Q1, -BIG)))

    s1_ref[0] = s1
    s2_ref[0] = s2
    m_ref[0] = m
    q1_ref[0] = Q1


def _head_body(s1_ref, s2_ref, m_ref, q1_ref, w2_ref,
               g1_ref, b1_ref, g2_ref, b2_ref, out_ref):
    s1 = s1_ref[...]
    s2 = s2_ref[...]
    m = m_ref[...]
    q1 = q1_ref[...]
    w2 = w2_ref[...]
    n = s1.shape[0]
    kf = jnp.float32(K)
    cnt1 = jnp.float32(n * K)

    sum1 = jnp.sum(s1 + kf * q1, axis=0, keepdims=True)
    sumsq = jnp.sum(s2 + 2.0 * q1 * s1 + kf * q1 * q1, axis=0, keepdims=True)
    mu1 = sum1 / cnt1
    var1 = sumsq / cnt1 - mu1 * mu1
    rs1 = lax.rsqrt(var1 + EPS)
    y = (m + q1 - mu1) * rs1 * g1_ref[...] + b1_ref[...]
    y = jnp.where(y >= 0.0, y, 0.2 * y)

    z = jnp.dot(y, w2.T, preferred_element_type=jnp.float32,
                precision=lax.Precision.HIGHEST)
    mu2 = jnp.sum(z, axis=0, keepdims=True) / n
    var2 = jnp.sum(z * z, axis=0, keepdims=True) / n - mu2 * mu2
    o = (z - mu2) * lax.rsqrt(var2 + EPS) * g2_ref[...] + b2_ref[...]
    out_ref[...] = jnp.where(o >= 0.0, o, 0.2 * o)


def kernel(query_features, query_xyz, key_value_features, key_value_xyz,
           W1, gamma1, beta1, W2, gamma2, beta2):
    B, Nq, C = query_features.shape
    Nkv = key_value_features.shape[1]
    Cd = W1.shape[0]
    nqb = Nq // BLOCK_Q
    kvxT = jnp.transpose(key_value_xyz, (0, 2, 1))

    Ab = pl.pallas_call(
        _proj_body,
        grid=(B,),
        in_specs=[
            pl.BlockSpec((1, Nkv, C), lambda b: (b, 0, 0)),
            pl.BlockSpec((Cd, 2 * C), lambda b: (0, 0)),
        ],
        out_specs=pl.BlockSpec((1, Nkv, 2 * Cd), lambda b: (b, 0, 0)),
        out_shape=jax.ShapeDtypeStruct((B, Nkv, 2 * Cd), jnp.bfloat16),
    )(key_value_features, W1)

    s1, s2, m, q1 = pl.pallas_call(
        _knn_body,
        grid=(B, nqb),
        in_specs=[
            pl.BlockSpec((1, BLOCK_Q, 3), lambda b, q: (b, q, 0)),
            pl.BlockSpec((1, 3, Nkv), lambda b, q: (b, 0, 0)),
            pl.BlockSpec((1, Nkv, 2 * Cd), lambda b, q: (b, 0, 0)),
            pl.BlockSpec((1, BLOCK_Q, C), lambda b, q: (b, q, 0)),
            pl.BlockSpec((Cd, 2 * C), lambda b, q: (0, 0)),
        ],
        out_specs=[
            pl.BlockSpec((1, BLOCK_Q, Cd), lambda b, q: (b, q, 0)),
            pl.BlockSpec((1, BLOCK_Q, Cd), lambda b, q: (b, q, 0)),
            pl.BlockSpec((1, BLOCK_Q, Cd), lambda b, q: (b, q, 0)),
            pl.BlockSpec((1, BLOCK_Q, Cd), lambda b, q: (b, q, 0)),
        ],
        out_shape=[jax.ShapeDtypeStruct((B, Nq, Cd), jnp.float32)] * 4,
        scratch_shapes=[pltpu.VMEM((BLOCK_Q, Nkv), jnp.float32)],
    )(query_xyz, kvxT, Ab, query_features, W1)

    n = B * Nq
    out = pl.pallas_call(
        _head_body,
        out_shape=jax.ShapeDtypeStruct((n, C), jnp.float32),
    )(s1.reshape(n, Cd), s2.reshape(n, Cd), m.reshape(n, Cd),
      q1.reshape(n, Cd), W2,
      gamma1.reshape(1, Cd), beta1.reshape(1, Cd),
      gamma2.reshape(1, C), beta2.reshape(1, C))

    return out.reshape(B, Nq, C)


# speedup vs baseline: 10.1269x; 1.0809x over previous
import jax
import jax.numpy as jnp
from jax import lax
from jax.experimental import pallas as pl
from jax.experimental.pallas import tpu as pltpu

K = 16
EPS = 1e-5
BLOCK_Q = 256
BIG = 3.0e38


def _proj_body(kvf_ref, w1_ref, ab_ref):
    w1 = w1_ref[...]
    C = kvf_ref.shape[2]
    w1a = w1[:, :C]
    A = jnp.dot(kvf_ref[0], w1a.T, preferred_element_type=jnp.float32,
                precision=lax.Precision.HIGHEST)
    hi = A.astype(jnp.bfloat16)
    lo = (A - hi.astype(jnp.float32)).astype(jnp.bfloat16)
    ab_ref[0] = jnp.concatenate([hi, lo], axis=1)


def _knn_body(qx_ref, kvxT_ref, ab_ref, qf_ref, w1_ref,
              s1_ref, s2_ref, m_ref, q1_ref, d_ref):
    qx = qx_ref[0]
    kvxT = kvxT_ref[0]
    Ab = ab_ref[0]
    qf = qf_ref[0]
    w1 = w1_ref[...]
    C = qf.shape[1]
    w1d = w1[:, C:] - w1[:, :C]
    Q1 = jnp.dot(qf, w1d.T, preferred_element_type=jnp.float32,
                 precision=lax.Precision.HIGHEST)

    def rq(x):
        return x.astype(jnp.bfloat16).astype(jnp.float32)

    qx0, qx1, qx2 = qx[:, 0:1], qx[:, 1:2], qx[:, 2:3]
    kx0, kx1, kx2 = kvxT[0:1, :], kvxT[1:2, :], kvxT[2:3, :]
    qk = rq(qx0) * rq(kx0) + rq(qx1) * rq(kx1) + rq(qx2) * rq(kx2)
    q2 = qx0 * qx0 + qx1 * qx1 + qx2 * qx2
    k2 = kx0 * kx0 + kx1 * kx1 + kx2 * kx2
    d_ref[...] = (-2.0 * qk + q2) + k2

    Cd = w1.shape[0]
    BQ, Nkv = d_ref.shape
    iota = lax.broadcasted_iota(jnp.int32, (BQ, Nkv), 1).astype(jnp.float32)

    def round_fn(_, carry):
        s1, s2, m = carry
        D = d_ref[...]
        v = jnp.min(D, axis=1, keepdims=True)
        cand = jnp.where(D == v, iota, BIG)
        jmin = jnp.min(cand, axis=1, keepdims=True)
        msk = cand == jmin
        row2 = jnp.dot(msk.astype(jnp.bfloat16), Ab,
                       preferred_element_type=jnp.float32)
        row = row2[:, :Cd] + row2[:, Cd:]
        d_ref[...] = jnp.where(msk, BIG, D)
        return s1 + row, s2 + row * row, jnp.maximum(m, row)

    s1, s2, m = lax.fori_loop(
        0, K,
        round_fn,
        (jnp.zeros_like(Q1), jnp.zeros_like(Q1), jnp.full_like(Q1, -BIG)))

    s1_ref[0] = s1
    s2_ref[0] = s2
    m_ref[0] = m
    q1_ref[0] = Q1


def _head_body(s1_ref, s2_ref, m_ref, q1_ref, w2_ref,
               g1_ref, b1_ref, g2_ref, b2_ref, out_ref):
    s1 = s1_ref[...]
    s2 = s2_ref[...]
    m = m_ref[...]
    q1 = q1_ref[...]
    w2 = w2_ref[...]
    n = s1.shape[0]
    kf = jnp.float32(K)
    cnt1 = jnp.float32(n * K)

    sum1 = jnp.sum(s1 + kf * q1, axis=0, keepdims=True)
    sumsq = jnp.sum(s2 + 2.0 * q1 * s1 + kf * q1 * q1, axis=0, keepdims=True)
    mu1 = sum1 / cnt1
    var1 = sumsq / cnt1 - mu1 * mu1
    rs1 = lax.rsqrt(var1 + EPS)
    y = (m + q1 - mu1) * rs1 * g1_ref[...] + b1_ref[...]
    y = jnp.where(y >= 0.0, y, 0.2 * y)

    z = jnp.dot(y, w2.T, preferred_element_type=jnp.float32,
                precision=lax.Precision.HIGHEST)
    mu2 = jnp.sum(z, axis=0, keepdims=True) / n
    var2 = jnp.sum(z * z, axis=0, keepdims=True) / n - mu2 * mu2
    o = (z - mu2) * lax.rsqrt(var2 + EPS) * g2_ref[...] + b2_ref[...]
    out_ref[...] = jnp.where(o >= 0.0, o, 0.2 * o)


def kernel(query_features, query_xyz, key_value_features, key_value_xyz,
           W1, gamma1, beta1, W2, gamma2, beta2):
    B, Nq, C = query_features.shape
    Nkv = key_value_features.shape[1]
    Cd = W1.shape[0]
    nqb = Nq // BLOCK_Q
    kvxT = jnp.transpose(key_value_xyz, (0, 2, 1))

    Ab = pl.pallas_call(
        _proj_body,
        grid=(B,),
        in_specs=[
            pl.BlockSpec((1, Nkv, C), lambda b: (b, 0, 0)),
            pl.BlockSpec((Cd, 2 * C), lambda b: (0, 0)),
        ],
        out_specs=pl.BlockSpec((1, Nkv, 2 * Cd), lambda b: (b, 0, 0)),
        out_shape=jax.ShapeDtypeStruct((B, Nkv, 2 * Cd), jnp.bfloat16),
    )(key_value_features, W1)

    s1, s2, m, q1 = pl.pallas_call(
        _knn_body,
        grid=(B, nqb),
        in_specs=[
            pl.BlockSpec((1, BLOCK_Q, 3), lambda b, q: (b, q, 0)),
            pl.BlockSpec((1, 3, Nkv), lambda b, q: (b, 0, 0)),
            pl.BlockSpec((1, Nkv, 2 * Cd), lambda b, q: (b, 0, 0)),
            pl.BlockSpec((1, BLOCK_Q, C), lambda b, q: (b, q, 0)),
            pl.BlockSpec((Cd, 2 * C), lambda b, q: (0, 0)),
        ],
        out_specs=[
            pl.BlockSpec((1, BLOCK_Q, Cd), lambda b, q: (b, q, 0)),
            pl.BlockSpec((1, BLOCK_Q, Cd), lambda b, q: (b, q, 0)),
            pl.BlockSpec((1, BLOCK_Q, Cd), lambda b, q: (b, q, 0)),
            pl.BlockSpec((1, BLOCK_Q, Cd), lambda b, q: (b, q, 0)),
        ],
        out_shape=[jax.ShapeDtypeStruct((B, Nq, Cd), jnp.float32)] * 4,
        scratch_shapes=[pltpu.VMEM((BLOCK_Q, Nkv), jnp.float32)],
    )(query_xyz, kvxT, Ab, query_features, W1)

    n = B * Nq
    out = pl.pallas_call(
        _head_body,
        out_shape=jax.ShapeDtypeStruct((n, C), jnp.float32),
    )(s1.reshape(n, Cd), s2.reshape(n, Cd), m.reshape(n, Cd),
      q1.reshape(n, Cd), W2,
      gamma1.reshape(1, Cd), beta1.reshape(1, Cd),
      gamma2.reshape(1, C), beta2.reshape(1, C))

    return out.reshape(B, Nq, C)


# speedup vs baseline: 10.5345x; 1.0402x over previous
import jax
import jax.numpy as jnp
from jax import lax
from jax.experimental import pallas as pl
from jax.experimental.pallas import tpu as pltpu

K = 16
EPS = 1e-5
BLOCK_Q = 512
BIG = 3.0e38


def _proj_body(kvf_ref, w1_ref, ab_ref):
    w1 = w1_ref[...]
    C = kvf_ref.shape[2]
    w1a = w1[:, :C]
    A = jnp.dot(kvf_ref[0], w1a.T, preferred_element_type=jnp.float32,
                precision=lax.Precision.HIGHEST)
    hi = A.astype(jnp.bfloat16)
    lo = (A - hi.astype(jnp.float32)).astype(jnp.bfloat16)
    ab_ref[0] = jnp.concatenate([hi, lo], axis=1)


def _knn_body(qx_ref, kvxT_ref, ab_ref, qf_ref, w1_ref,
              s1_ref, s2_ref, m_ref, q1_ref, d_ref):
    qx = qx_ref[0]
    kvxT = kvxT_ref[0]
    Ab = ab_ref[0]
    qf = qf_ref[0]
    w1 = w1_ref[...]
    C = qf.shape[1]
    w1d = w1[:, C:] - w1[:, :C]
    Q1 = jnp.dot(qf, w1d.T, preferred_element_type=jnp.float32,
                 precision=lax.Precision.HIGHEST)

    def rq(x):
        return x.astype(jnp.bfloat16).astype(jnp.float32)

    qx0, qx1, qx2 = qx[:, 0:1], qx[:, 1:2], qx[:, 2:3]
    kx0, kx1, kx2 = kvxT[0:1, :], kvxT[1:2, :], kvxT[2:3, :]
    qk = rq(qx0) * rq(kx0) + rq(qx1) * rq(kx1) + rq(qx2) * rq(kx2)
    q2 = qx0 * qx0 + qx1 * qx1 + qx2 * qx2
    k2 = kx0 * kx0 + kx1 * kx1 + kx2 * kx2
    d_ref[...] = (-2.0 * qk + q2) + k2

    Cd = w1.shape[0]
    BQ, Nkv = d_ref.shape
    iota = lax.broadcasted_iota(jnp.int32, (BQ, Nkv), 1).astype(jnp.float32)

    def round_fn(_, carry):
        s1, s2, m = carry
        D = d_ref[...]
        v = jnp.min(D, axis=1, keepdims=True)
        cand = jnp.where(D == v, iota, BIG)
        jmin = jnp.min(cand, axis=1, keepdims=True)
        msk = cand == jmin
        row2 = jnp.dot(msk.astype(jnp.bfloat16), Ab,
                       preferred_element_type=jnp.float32)
        row = row2[:, :Cd] + row2[:, Cd:]
        d_ref[...] = jnp.where(msk, BIG, D)
        return s1 + row, s2 + row * row, jnp.maximum(m, row)

    s1, s2, m = lax.fori_loop(
        0, K,
        round_fn,
        (jnp.zeros_like(Q1), jnp.zeros_like(Q1), jnp.full_like(Q1, -BIG)))

    s1_ref[0] = s1
    s2_ref[0] = s2
    m_ref[0] = m
    q1_ref[0] = Q1


def _head_body(s1_ref, s2_ref, m_ref, q1_ref, w2_ref,
               g1_ref, b1_ref, g2_ref, b2_ref, out_ref):
    s1 = s1_ref[...]
    s2 = s2_ref[...]
    m = m_ref[...]
    q1 = q1_ref[...]
    w2 = w2_ref[...]
    n = s1.shape[0]
    kf = jnp.float32(K)
    cnt1 = jnp.float32(n * K)

    sum1 = jnp.sum(s1 + kf * q1, axis=0, keepdims=True)
    sumsq = jnp.sum(s2 + 2.0 * q1 * s1 + kf * q1 * q1, axis=0, keepdims=True)
    mu1 = sum1 / cnt1
    var1 = sumsq / cnt1 - mu1 * mu1
    rs1 = lax.rsqrt(var1 + EPS)
    y = (m + q1 - mu1) * rs1 * g1_ref[...] + b1_ref[...]
    y = jnp.where(y >= 0.0, y, 0.2 * y)

    z = jnp.dot(y, w2.T, preferred_element_type=jnp.float32,
                precision=lax.Precision.HIGHEST)
    mu2 = jnp.sum(z, axis=0, keepdims=True) / n
    var2 = jnp.sum(z * z, axis=0, keepdims=True) / n - mu2 * mu2
    o = (z - mu2) * lax.rsqrt(var2 + EPS) * g2_ref[...] + b2_ref[...]
    out_ref[...] = jnp.where(o >= 0.0, o, 0.2 * o)


def kernel(query_features, query_xyz, key_value_features, key_value_xyz,
           W1, gamma1, beta1, W2, gamma2, beta2):
    B, Nq, C = query_features.shape
    Nkv = key_value_features.shape[1]
    Cd = W1.shape[0]
    nqb = Nq // BLOCK_Q
    kvxT = jnp.transpose(key_value_xyz, (0, 2, 1))

    Ab = pl.pallas_call(
        _proj_body,
        grid=(B,),
        in_specs=[
            pl.BlockSpec((1, Nkv, C), lambda b: (b, 0, 0)),
            pl.BlockSpec((Cd, 2 * C), lambda b: (0, 0)),
        ],
        out_specs=pl.BlockSpec((1, Nkv, 2 * Cd), lambda b: (b, 0, 0)),
        out_shape=jax.ShapeDtypeStruct((B, Nkv, 2 * Cd), jnp.bfloat16),
    )(key_value_features, W1)

    s1, s2, m, q1 = pl.pallas_call(
        _knn_body,
        grid=(B, nqb),
        in_specs=[
            pl.BlockSpec((1, BLOCK_Q, 3), lambda b, q: (b, q, 0)),
            pl.BlockSpec((1, 3, Nkv), lambda b, q: (b, 0, 0)),
            pl.BlockSpec((1, Nkv, 2 * Cd), lambda b, q: (b, 0, 0)),
            pl.BlockSpec((1, BLOCK_Q, C), lambda b, q: (b, q, 0)),
            pl.BlockSpec((Cd, 2 * C), lambda b, q: (0, 0)),
        ],
        out_specs=[
            pl.BlockSpec((1, BLOCK_Q, Cd), lambda b, q: (b, q, 0)),
            pl.BlockSpec((1, BLOCK_Q, Cd), lambda b, q: (b, q, 0)),
            pl.BlockSpec((1, BLOCK_Q, Cd), lambda b, q: (b, q, 0)),
            pl.BlockSpec((1, BLOCK_Q, Cd), lambda b, q: (b, q, 0)),
        ],
        out_shape=[jax.ShapeDtypeStruct((B, Nq, Cd), jnp.float32)] * 4,
        scratch_shapes=[pltpu.VMEM((BLOCK_Q, Nkv), jnp.float32)],
    )(query_xyz, kvxT, Ab, query_features, W1)

    n = B * Nq
    out = pl.pallas_call(
        _head_body,
        out_shape=jax.ShapeDtypeStruct((n, C), jnp.float32),
    )(s1.reshape(n, Cd), s2.reshape(n, Cd), m.reshape(n, Cd),
      q1.reshape(n, Cd), W2,
      gamma1.reshape(1, Cd), beta1.reshape(1, Cd),
      gamma2.reshape(1, C), beta2.reshape(1, C))

    return out.reshape(B, Nq, C)
